# baseline (speedup 1.0000x reference)
import functools

import jax
import jax.numpy as jnp
from jax import lax
from jax.experimental import pallas as pl
from jax.experimental.pallas import tpu as pltpu
from jax.experimental.pallas import tpu_sc as plsc

_NC = 2
_NS = 16
_NW = _NC * _NS

_BC = 32
_SCK = 40
_LN = 16

_MESH = plsc.VectorSubcoreMesh(core_axis_name="c", subcore_axis_name="s")
_PARAMS = pltpu.CompilerParams(
    use_tc_tiling_on_sc=False, needs_layout_passes=False)
_PARAMS_TILED = pltpu.CompilerParams(
    use_tc_tiling_on_sc=True, needs_layout_passes=False)


def _make_detile_call(V, D):
    KR = 1024
    PIT = 1029
    n_sb = (V - 576) // KR
    assert n_sb * KR + 576 == V

    @functools.partial(
        pl.kernel,
        mesh=_MESH,
        compiler_params=_PARAMS_TILED,
        out_type=jax.ShapeDtypeStruct((V * D // 128, 128), jnp.float32),
        scratch_types=[
            pltpu.VMEM((D // 8, 8, PIT), jnp.float32),
            pltpu.VMEM((KR * D // 128, 128), jnp.float32),
            pltpu.SemaphoreType.DMA,
            pltpu.SemaphoreType.DMA,
        ],
    )
    def detile_call(tab3_hbm, tail_hbm, t128_hbm, in_v, out_v, isem, osem):
        wid = lax.axis_index("s") * _NC + lax.axis_index("c")
        n_mine = 30 + jnp.where(wid < 16, 1, 0)

        dd = lax.iota(jnp.int32, _LN)
        t_c = [(h * _LN + dd) // 8 for h in range(D // _LN)]
        i_c = [(h * _LN + dd) % 8 for h in range(D // _LN)]

        def transpose_rows(n_rows):
            def row_body(r, carry):
                for q in range(4):
                    jj = jnp.full((_LN,), 0, jnp.int32) + (r * 4 + q)
                    for h in range(D // _LN):
                        v = plsc.load_gather(in_v, [t_c[h], i_c[h], jj])
                        out_v[r, pl.ds(q * 32 + h * _LN, _LN)] = v
                return carry
            lax.fori_loop(0, n_rows, row_body, 0)

        def stage_in(r0, nr):
            descs = []
            for t in range(D // 8):
                descs.append(pltpu.async_copy(
                    tab3_hbm.at[t, :, pl.ds(r0, nr)],
                    in_v.at[t, :, pl.ds(0, nr)], isem))
            for dsc in descs:
                dsc.wait()

        def sb_body(i, carry):
            sb = wid + i * _NW
            r0 = pl.multiple_of(sb * KR, KR)
            stage_in(r0, KR)
            transpose_rows(KR // 4)
            pltpu.async_copy(
                out_v, t128_hbm.at[pl.ds(sb * (KR * D // 128),
                                         KR * D // 128)], osem).wait()
            return carry
        lax.fori_loop(0, n_mine, sb_body, 0)

        @pl.when(wid == 0)
        def _tail0():
            stage_in(n_sb * KR, 512)
            transpose_rows(128)
            pltpu.async_copy(
                out_v.at[pl.ds(0, 128)],
                t128_hbm.at[pl.ds(n_sb * (KR * D // 128), 128)],
                osem).wait()

        @pl.when(wid == 1)
        def _tail1():
            pltpu.async_copy(tail_hbm, out_v.at[pl.ds(0, 16)], isem).wait()
            pltpu.async_copy(
                out_v.at[pl.ds(0, 16)],
                t128_hbm.at[pl.ds(n_sb * (KR * D // 128) + 128, 16)],
                osem).wait()

    return detile_call


def _make_gather_call(B, S, V, D):
    b_per_w = B // _NW
    nb = b_per_w // _BC
    ns = S // _SCK
    dh_n = D // 8

    @functools.partial(
        pl.kernel,
        mesh=_MESH,
        compiler_params=_PARAMS,
        out_type=jax.ShapeDtypeStruct((S, dh_n, _NW, 8, 128), jnp.float32),
        scratch_types=[
            pltpu.VMEM((_BC, _SCK), jnp.int32),
            pltpu.VMEM((_BC * _SCK, D), jnp.float32),
            pltpu.VMEM((_SCK, dh_n, 8, _BC + 1), jnp.float32),
            pltpu.VMEM((S, D), jnp.float32),
            pltpu.SemaphoreType.DMA,
            pltpu.SemaphoreType.DMA,
        ],
    )
    def gather_call(x_hbm, tab_hbm, pos_hbm, out_hbm,
                    idx_v, rows_v, trans_v, pos_v, gsem, msem):
        wid = lax.axis_index("s") * _NC + lax.axis_index("c")
        b_base = wid * b_per_w

        pltpu.async_copy(pos_hbm, pos_v, msem).wait()

        def chunk_body(g, carry):
            bl0 = pl.multiple_of((g // ns) * _BC, _BC)
            s0 = pl.multiple_of((g % ns) * _SCK, _SCK)

            pltpu.async_copy(
                x_hbm.at[pl.ds(b_base + bl0, _BC), pl.ds(s0, _SCK)],
                idx_v, msem,
            ).wait()

            descs = []
            for j in range(_BC):
                descs.append(pltpu.async_copy(
                    tab_hbm.at[idx_v.at[j]],
                    rows_v.at[pl.ds(j * _SCK, _SCK)],
                    gsem,
                ))
            for dsc in descs:
                dsc.wait()

            dd = lax.iota(jnp.int32, _LN)
            dh_c = [(dd + h * _LN) // 8 for h in range(D // _LN)]
            dl_c = [(dd + h * _LN) % 8 for h in range(D // _LN)]
            def tr_body(s, carry2):
                s_vec = jnp.full((_LN,), 0, jnp.int32) + s
                pos_h = [pos_v[s0 + s, pl.ds(h * _LN, _LN)]
                         for h in range(D // _LN)]
                for j in range(_BC):
                    r = j * _SCK + s
                    j_vec = jnp.full((_LN,), j, dtype=jnp.int32)
                    for h in range(D // _LN):
                        v = rows_v[r, pl.ds(h * _LN, _LN)] + pos_h[h]
                        plsc.store_scatter(
                            trans_v, [s_vec, dh_c[h], dl_c[h], j_vec], v)
                return carry2
            lax.fori_loop(0, _SCK, tr_body, 0)

            for dh in range(dh_n):
                pltpu.async_copy(
                    trans_v.at[:, dh, :, pl.ds(0, _BC)],
                    out_hbm.at[pl.ds(s0, _SCK), dh, wid, :,
                               pl.ds(bl0, _BC)],
                    msem,
                ).wait()
            return carry

        lax.fori_loop(0, nb * ns, chunk_body, 0)

    return gather_call


def kernel(x, table, pos_encoding):
    B, S = x.shape
    V, D = table.shape
    pos2d = pos_encoding[0, :S, :]
    tab3 = table.T.reshape(D // 8, 8, V)
    tail16 = table[V - 64:, :].reshape(16, 128)
    t128 = _make_detile_call(V, D)(tab3, tail16)
    table_lin = t128.reshape(V, D)
    out5 = _make_gather_call(B, S, V, D)(x, table_lin, pos2d)
    out = out5.transpose(2, 4, 0, 1, 3).reshape(B, S, D)
    return out


# speedup vs baseline: 1.4777x; 1.4777x over previous
import functools

import jax
import jax.numpy as jnp
from jax import lax
from jax.experimental import pallas as pl
from jax.experimental.pallas import tpu as pltpu
from jax.experimental.pallas import tpu_sc as plsc

_NC = 2
_NS = 16
_NW = _NC * _NS

_BC = 16
_SCK = 40
_LN = 16

_MESH = plsc.VectorSubcoreMesh(core_axis_name="c", subcore_axis_name="s")
_PARAMS = pltpu.CompilerParams(
    use_tc_tiling_on_sc=False, needs_layout_passes=False)


def _make_gather_call(B, S, V, D):
    b_per_w = B // _NW
    nb = b_per_w // _BC
    ns = S // _SCK
    dh_n = D // 8

    @functools.partial(
        pl.kernel,
        mesh=_MESH,
        compiler_params=_PARAMS,
        out_type=jax.ShapeDtypeStruct((S, dh_n, _NW, 8, 128), jnp.float32),
        scratch_types=[
            pltpu.VMEM((_BC, S), jnp.int32),
            [pltpu.VMEM((_BC * _SCK, D), jnp.float32)
             for _ in range(2)],
            [pltpu.VMEM((_SCK, dh_n, 8, _BC + 1), jnp.float32)
             for _ in range(2)],
            pltpu.VMEM((S, D), jnp.float32),
            pltpu.SemaphoreType.DMA,
            pltpu.SemaphoreType.DMA,
            pltpu.SemaphoreType.DMA,
        ],
    )
    def gather_call(x_hbm, tab_hbm, pos_hbm, out_hbm,
                    idx_v, rows_v, trans_v, pos_v, gsem, osem, msem):
        wid = lax.axis_index("s") * _NC + lax.axis_index("c")
        b_base = wid * b_per_w

        pltpu.async_copy(pos_hbm, pos_v, msem).wait()

        dd = lax.iota(jnp.int32, _LN)
        dh_c = [(dd + h * _LN) // 8 for h in range(D // _LN)]
        dl_c = [(dd + h * _LN) % 8 for h in range(D // _LN)]

        def fire_gathers(j_chunk):
            rv = rows_v[j_chunk % 2]
            s0 = j_chunk * _SCK
            return [pltpu.async_copy(
                tab_hbm.at[idx_v.at[j, pl.ds(s0, _SCK)]],
                rv.at[pl.ds(j * _SCK, _SCK)], gsem)
                for j in range(_BC)]

        def compute(j_chunk):
            rv, tv = rows_v[j_chunk % 2], trans_v[j_chunk % 2]
            s0 = j_chunk * _SCK
            def tr_body(s, carry2):
                s_vec = jnp.full((_LN,), 0, jnp.int32) + s
                pos_h = [pos_v[s0 + s, pl.ds(h * _LN, _LN)]
                         for h in range(D // _LN)]
                for j in range(_BC):
                    r = j * _SCK + s
                    j_vec = jnp.full((_LN,), j, dtype=jnp.int32)
                    for h in range(D // _LN):
                        v = rv[r, pl.ds(h * _LN, _LN)] + pos_h[h]
                        plsc.store_scatter(
                            tv, [s_vec, dh_c[h], dl_c[h], j_vec], v)
                return carry2
            lax.fori_loop(0, _SCK, tr_body, 0)

        def fire_out(j_chunk, bl0):
            tv = trans_v[j_chunk % 2]
            s0 = j_chunk * _SCK
            return [pltpu.async_copy(
                tv.at[:, dh, :, pl.ds(0, _BC)],
                out_hbm.at[pl.ds(s0, _SCK), dh, wid, :, pl.ds(bl0, _BC)],
                osem)
                for dh in range(dh_n)]

        def group_body(g, carry):
            bl0 = pl.multiple_of(g * _BC, _BC)
            pltpu.async_copy(
                x_hbm.at[pl.ds(b_base + bl0, _BC)], idx_v, msem).wait()

            gd = {0: fire_gathers(0)}
            od = {}
            for j in range(ns):
                if j + 1 < ns:
                    gd[j + 1] = fire_gathers(j + 1)
                if j >= 2:
                    for dsc in od.pop(j - 2):
                        dsc.wait()
                for dsc in gd.pop(j):
                    dsc.wait()
                compute(j)
                od[j] = fire_out(j, bl0)
            for j in (ns - 2, ns - 1):
                for dsc in od.pop(j):
                    dsc.wait()
            return carry

        lax.fori_loop(0, nb, group_body, 0)

    return gather_call


def kernel(x, table, pos_encoding):
    B, S = x.shape
    V, D = table.shape
    pos2d = pos_encoding[0, :S, :]
    out5 = _make_gather_call(B, S, V, D)(x, table, pos2d)
    out = out5.transpose(2, 4, 0, 1, 3).reshape(B, S, D)
    return out


# speedup vs baseline: 1.4793x; 1.0010x over previous
import functools

import jax
import jax.numpy as jnp
from jax import lax
from jax.experimental import pallas as pl
from jax.experimental.pallas import tpu as pltpu
from jax.experimental.pallas import tpu_sc as plsc

_NC = 2
_NS = 16
_NW = _NC * _NS

_BC = 16
_SCK = 40
_LN = 16

_MESH = plsc.VectorSubcoreMesh(core_axis_name="c", subcore_axis_name="s")
_PARAMS = pltpu.CompilerParams(
    use_tc_tiling_on_sc=False, needs_layout_passes=False)


def _make_gather_call(B, S, V, D):
    b_per_w = B // _NW
    nb = b_per_w // _BC
    ns = S // _SCK
    dh_n = D // 8

    @functools.partial(
        pl.kernel,
        mesh=_MESH,
        compiler_params=_PARAMS,
        out_type=jax.ShapeDtypeStruct((S, dh_n, _NW, 8, 128), jnp.float32),
        scratch_types=[
            pltpu.VMEM((_BC, S), jnp.int32),
            [pltpu.VMEM((_BC * _SCK, D), jnp.float32)
             for _ in range(2)],
            [pltpu.VMEM((_SCK, dh_n, 8, _BC + 1), jnp.float32)
             for _ in range(2)],
            pltpu.VMEM((S, D), jnp.float32),
            [pltpu.SemaphoreType.DMA for _ in range(2)],
            [pltpu.SemaphoreType.DMA for _ in range(2)],
            pltpu.SemaphoreType.DMA,
        ],
    )
    def gather_call(x_hbm, tab_hbm, pos_hbm, out_hbm,
                    idx_v, rows_v, trans_v, pos_v, gsem, osem, msem):
        wid = lax.axis_index("s") * _NC + lax.axis_index("c")
        b_base = wid * b_per_w

        pltpu.async_copy(pos_hbm, pos_v, msem).wait()

        dd = lax.iota(jnp.int32, _LN)
        dh_c = [(dd + h * _LN) // 8 for h in range(D // _LN)]
        dl_c = [(dd + h * _LN) % 8 for h in range(D // _LN)]

        def fire_gathers(j_chunk):
            rv = rows_v[j_chunk % 2]
            s0 = j_chunk * _SCK
            return [pltpu.async_copy(
                tab_hbm.at[idx_v.at[j, pl.ds(s0, _SCK)]],
                rv.at[pl.ds(j * _SCK, _SCK)], gsem[j_chunk % 2])
                for j in range(_BC)]

        def compute(j_chunk):
            rv, tv = rows_v[j_chunk % 2], trans_v[j_chunk % 2]
            s0 = j_chunk * _SCK
            def tr_body(s, carry2):
                s_vec = jnp.full((_LN,), 0, jnp.int32) + s
                pos_h = [pos_v[s0 + s, pl.ds(h * _LN, _LN)]
                         for h in range(D // _LN)]
                for j in range(_BC):
                    r = j * _SCK + s
                    j_vec = jnp.full((_LN,), j, dtype=jnp.int32)
                    for h in range(D // _LN):
                        v = rv[r, pl.ds(h * _LN, _LN)] + pos_h[h]
                        plsc.store_scatter(
                            tv, [s_vec, dh_c[h], dl_c[h], j_vec], v)
                return carry2
            lax.fori_loop(0, _SCK, tr_body, 0)

        def fire_out(j_chunk, bl0):
            tv = trans_v[j_chunk % 2]
            s0 = j_chunk * _SCK
            return [pltpu.async_copy(
                tv.at[:, dh, :, pl.ds(0, _BC)],
                out_hbm.at[pl.ds(s0, _SCK), dh, wid, :, pl.ds(bl0, _BC)],
                osem[j_chunk % 2])
                for dh in range(dh_n)]

        def group_body(g, carry):
            bl0 = pl.multiple_of(g * _BC, _BC)
            pltpu.async_copy(
                x_hbm.at[pl.ds(b_base + bl0, _BC)], idx_v, msem).wait()

            gd = {0: fire_gathers(0)}
            od = {}
            for j in range(ns):
                if j + 1 < ns:
                    gd[j + 1] = fire_gathers(j + 1)
                if j >= 2:
                    for dsc in od.pop(j - 2):
                        dsc.wait()
                for dsc in gd.pop(j):
                    dsc.wait()
                compute(j)
                od[j] = fire_out(j, bl0)
            for j in (ns - 2, ns - 1):
                for dsc in od.pop(j):
                    dsc.wait()
            return carry

        lax.fori_loop(0, nb, group_body, 0)

    return gather_call


def kernel(x, table, pos_encoding):
    B, S = x.shape
    V, D = table.shape
    pos2d = pos_encoding[0, :S, :]
    out5 = _make_gather_call(B, S, V, D)(x, table, pos2d)
    out = out5.transpose(2, 4, 0, 1, 3).reshape(B, S, D)
    return out


# speedup vs baseline: 1.8257x; 1.2342x over previous
import functools

import jax
import jax.numpy as jnp
from jax import lax
from jax.experimental import pallas as pl
from jax.experimental.pallas import tpu as pltpu
from jax.experimental.pallas import tpu_sc as plsc

_NC = 2
_NS = 16
_NW = _NC * _NS

_BC = 16
_SCK = 40
_LN = 16

_MESH = plsc.VectorSubcoreMesh(core_axis_name="c", subcore_axis_name="s")
_PARAMS = pltpu.CompilerParams(
    use_tc_tiling_on_sc=False, needs_layout_passes=False)


def _make_gather_call(B, S, V, D):
    b_per_w = B // _NW
    nb = b_per_w // _BC
    ns = S // _SCK
    dh_n = D // 8

    @functools.partial(
        pl.kernel,
        mesh=_MESH,
        compiler_params=_PARAMS,
        out_type=jax.ShapeDtypeStruct((S, dh_n, _NW, 8, 128), jnp.float32),
        scratch_types=[
            pltpu.VMEM((_BC, S), jnp.int32),
            [pltpu.VMEM((_BC * _SCK, D), jnp.float32)
             for _ in range(2)],
            [pltpu.VMEM((_SCK, dh_n, 8, _BC + 1), jnp.float32)
             for _ in range(2)],
            pltpu.VMEM((S, D), jnp.float32),
            [pltpu.SemaphoreType.DMA for _ in range(2)],
            [pltpu.SemaphoreType.DMA for _ in range(2)],
            pltpu.SemaphoreType.DMA,
        ],
    )
    def gather_call(x_hbm, tab_hbm, pos_hbm, out_hbm,
                    idx_v, rows_v, trans_v, pos_v, gsem, osem, msem):
        wid = lax.axis_index("s") * _NC + lax.axis_index("c")
        b_base = wid * b_per_w

        pltpu.async_copy(pos_hbm, pos_v, msem).wait()

        dd = lax.iota(jnp.int32, _LN)
        dh_c = [(dd + h * _LN) // 8 for h in range(D // _LN)]
        dl_c = [(dd + h * _LN) % 8 for h in range(D // _LN)]

        def fire_gathers(j_chunk):
            rv = rows_v[j_chunk % 2]
            s0 = j_chunk * _SCK
            return [pltpu.async_copy(
                tab_hbm.at[idx_v.at[j, pl.ds(s0, _SCK)]],
                rv.at[pl.ds(j * _SCK, _SCK)], gsem[j_chunk % 2])
                for j in range(_BC)]

        def compute(j_chunk):
            rv, tv = rows_v[j_chunk % 2], trans_v[j_chunk % 2]
            s0 = j_chunk * _SCK

            @plsc.parallel_loop(0, _SCK, unroll=2)
            def tr_body(s):
                s_vec = jnp.full((_LN,), 0, jnp.int32) + s
                pos_h = [pos_v[s0 + s, pl.ds(h * _LN, _LN)]
                         for h in range(D // _LN)]
                for j in range(_BC):
                    r = j * _SCK + s
                    j_vec = jnp.full((_LN,), j, dtype=jnp.int32)
                    for h in range(D // _LN):
                        v = rv[r, pl.ds(h * _LN, _LN)] + pos_h[h]
                        plsc.store_scatter(
                            tv, [s_vec, dh_c[h], dl_c[h], j_vec], v)

        def fire_out(j_chunk, bl0):
            tv = trans_v[j_chunk % 2]
            s0 = j_chunk * _SCK
            return [pltpu.async_copy(
                tv.at[:, dh, :, pl.ds(0, _BC)],
                out_hbm.at[pl.ds(s0, _SCK), dh, wid, :, pl.ds(bl0, _BC)],
                osem[j_chunk % 2])
                for dh in range(dh_n)]

        def group_body(g, carry):
            bl0 = pl.multiple_of(g * _BC, _BC)
            pltpu.async_copy(
                x_hbm.at[pl.ds(b_base + bl0, _BC)], idx_v, msem).wait()

            gd = {0: fire_gathers(0)}
            od = {}
            for j in range(ns):
                if j + 1 < ns:
                    gd[j + 1] = fire_gathers(j + 1)
                if j >= 2:
                    for dsc in od.pop(j - 2):
                        dsc.wait()
                for dsc in gd.pop(j):
                    dsc.wait()
                compute(j)
                od[j] = fire_out(j, bl0)
            for j in (ns - 2, ns - 1):
                for dsc in od.pop(j):
                    dsc.wait()
            return carry

        lax.fori_loop(0, nb, group_body, 0)

    return gather_call


def kernel(x, table, pos_encoding):
    B, S = x.shape
    V, D = table.shape
    pos2d = pos_encoding[0, :S, :]
    out5 = _make_gather_call(B, S, V, D)(x, table, pos2d)
    out = out5.transpose(2, 4, 0, 1, 3).reshape(B, S, D)
    return out
